# baseline (speedup 1.0000x reference)
import dataclasses
import functools

import jax
import jax.numpy as jnp
from jax import lax
from jax.experimental import pallas as pl
from jax.experimental.pallas import tpu as pltpu
from jax.experimental.pallas import tpu_sc as plsc

L = 16
NC = 2
NS = 16
NW = NC * NS

EMBED = 16
HIDDEN = 8


def _scores_kernel(B: int):
    b_per_w = B // NW
    nblk = b_per_w // L
    mesh = plsc.VectorSubcoreMesh(core_axis_name="c", subcore_axis_name="s")
    cp = pltpu.CompilerParams(
        needs_layout_passes=False, use_tc_tiling_on_sc=False
    )

    @functools.partial(
        pl.kernel,
        mesh=mesh,
        compiler_params=cp,
        out_type=jax.ShapeDtypeStruct((B,), jnp.float32),
        scratch_types=[
            pltpu.VMEM((b_per_w,), jnp.int32),
            pltpu.VMEM((b_per_w, EMBED), jnp.float32),
            pltpu.VMEM((EMBED * HIDDEN + 2 * HIDDEN + 1, L), jnp.float32),
            pltpu.VMEM((b_per_w,), jnp.float32),
            pltpu.SemaphoreType.DMA,
        ],
    )
    def k(ids_hbm, table_hbm, w_hbm, out_hbm, idx_v, rows_v, w_v, score_v, sem):
        wid = lax.axis_index("s") * NC + lax.axis_index("c")
        base = wid * b_per_w
        pltpu.sync_copy(w_hbm, w_v)
        pltpu.sync_copy(ids_hbm.at[pl.ds(base, b_per_w)], idx_v)
        pltpu.async_copy(table_hbm.at[idx_v], rows_v, sem).wait()

        lanes = lax.iota(jnp.int32, L)

        @pl.loop(0, nblk)
        def _(i):
            row0 = i * L
            ridx = row0 + lanes
            cols = [
                plsc.load_gather(rows_v, [ridx, jnp.full((L,), kk, jnp.int32)])
                for kk in range(EMBED)
            ]
            score = w_v[EMBED * HIDDEN + 2 * HIDDEN]
            for j in range(HIDDEN):
                acc = w_v[EMBED * HIDDEN + j]
                for kk in range(EMBED):
                    acc = acc + cols[kk] * w_v[kk * HIDDEN + j]
                h = jnp.maximum(acc, 0.0)
                score = score + h * w_v[EMBED * HIDDEN + HIDDEN + j]
            score_v[pl.ds(row0, L)] = score

        pltpu.sync_copy(score_v, out_hbm.at[pl.ds(base, b_per_w)])

    return k


def kernel(title_ids, table, W1, b1, W2, b2):
    B = title_ids.shape[0]
    w1b = jnp.broadcast_to(W1.reshape(EMBED, HIDDEN, 1), (EMBED, HIDDEN, L))
    w1b = w1b.reshape(EMBED * HIDDEN, L)
    b1b = jnp.broadcast_to(b1.reshape(HIDDEN, 1), (HIDDEN, L))
    w2b = jnp.broadcast_to(W2.reshape(HIDDEN, 1), (HIDDEN, L))
    b2b = jnp.broadcast_to(b2.reshape(1, 1), (1, L))
    wall = jnp.concatenate([w1b, b1b, w2b, b2b], axis=0).astype(jnp.float32)

    scores = _scores_kernel(B)(title_ids.astype(jnp.int32), table, wall)
    return scores.reshape(B, 1)
